# baseline (speedup 1.0000x reference)
import functools

import jax
import jax.numpy as jnp
from jax.experimental import pallas as pl
from jax.experimental.pallas import tpu as pltpu
from jax.experimental.pallas import tpu_sc as plsc

_Q_CONST = 1000.0
_VPD_CONST = 1.5
_LANES = 16


def kernel(Em, i0, k, b, FGs):
    n_fg = Em.shape[0]
    n_out = FGs.shape[0]

    mesh = plsc.VectorSubcoreMesh(
        core_axis_name="c", subcore_axis_name="s",
        num_cores=1, num_subcores=1)

    @functools.partial(
        pl.kernel,
        mesh=mesh,
        out_type=jax.ShapeDtypeStruct((n_out,), jnp.float32),
        scratch_types=[
            pltpu.VMEM((_LANES,), jnp.float32),
            pltpu.VMEM((_LANES,), jnp.float32),
            pltpu.VMEM((_LANES,), jnp.float32),
            pltpu.VMEM((_LANES,), jnp.float32),
            pltpu.VMEM((_LANES,), jnp.float32),
            pltpu.SemaphoreType.DMA,
        ],
    )
    def _bmf(em_h, i0_h, k_h, b_h, fg_h, out_h,
             em_v, i0_v, k_v, b_v, out_v, sem):
        del fg_h
        cps = (
            pltpu.async_copy(em_h, em_v.at[pl.ds(0, n_fg)], sem),
            pltpu.async_copy(i0_h, i0_v.at[pl.ds(0, n_fg)], sem),
            pltpu.async_copy(k_h, k_v.at[pl.ds(0, n_fg)], sem),
            pltpu.async_copy(b_h, b_v.at[pl.ds(0, n_fg)], sem),
        )
        for cp in cps:
            cp.wait()
        qi = _Q_CONST + i0_v[...]
        out_v[...] = em_v[...] * qi / (k_v[...] + b_v[...] * _Q_CONST
                                       + qi * _VPD_CONST)
        pltpu.sync_copy(out_v.at[pl.ds(0, n_out)], out_h)

    return _bmf(Em, i0, k, b, FGs)


# speedup vs baseline: 1.0259x; 1.0259x over previous
import functools

import jax
import jax.numpy as jnp
from jax import lax
from jax.experimental import pallas as pl
from jax.experimental.pallas import tpu as pltpu
from jax.experimental.pallas import tpu_sc as plsc

_Q_CONST = 1000.0
_VPD_CONST = 1.5
_LANES = 16


def kernel(Em, i0, k, b, FGs):
    n_fg = Em.shape[0]
    n_out = FGs.shape[0]

    mesh = plsc.VectorSubcoreMesh(
        core_axis_name="c", subcore_axis_name="s",
        num_cores=1, num_subcores=1)

    @functools.partial(
        pl.kernel,
        mesh=mesh,
        out_type=jax.ShapeDtypeStruct((n_out,), jnp.float32),
        scratch_types=[
            pltpu.VMEM((_LANES,), jnp.float32),
            pltpu.VMEM((_LANES,), jnp.float32),
            pltpu.VMEM((_LANES,), jnp.float32),
            pltpu.VMEM((_LANES,), jnp.float32),
            pltpu.VMEM((_LANES,), jnp.int32),
            pltpu.VMEM((_LANES,), jnp.float32),
            pltpu.SemaphoreType.DMA,
        ],
        compiler_params=pltpu.CompilerParams(skip_device_barrier=True),
    )
    def _bmf(em_h, i0_h, k_h, b_h, fg_h, out_h,
             em_v, i0_v, k_v, b_v, fg_v, out_v, sem):
        cps = (
            pltpu.async_copy(em_h, em_v.at[pl.ds(0, n_fg)], sem),
            pltpu.async_copy(i0_h, i0_v.at[pl.ds(0, n_fg)], sem),
            pltpu.async_copy(k_h, k_v.at[pl.ds(0, n_fg)], sem),
            pltpu.async_copy(b_h, b_v.at[pl.ds(0, n_fg)], sem),
            pltpu.async_copy(fg_h, fg_v.at[pl.ds(0, n_out)], sem),
        )
        for cp in cps:
            cp.wait()
        idx = jnp.clip(fg_v[...], 0, n_fg - 1)

        dnums = lax.GatherDimensionNumbers(
            offset_dims=(), collapsed_slice_dims=(0,),
            start_index_map=(0,))

        def _gather(vec_ref):
            return lax.gather(
                vec_ref[...], idx.reshape(_LANES, 1), dnums, (1,),
                mode=lax.GatherScatterMode.PROMISE_IN_BOUNDS)

        em_g = _gather(em_v)
        i0_g = _gather(i0_v)
        k_g = _gather(k_v)
        b_g = _gather(b_v)
        qi = _Q_CONST + i0_g
        out_v[...] = em_g * qi / (k_g + b_g * _Q_CONST + qi * _VPD_CONST)
        pltpu.sync_copy(out_v.at[pl.ds(0, n_out)], out_h)

    return _bmf(Em, i0, k, b, FGs.astype(jnp.int32))
